# baseline (speedup 1.0000x reference)
import functools

import jax
import jax.numpy as jnp
from jax import lax
from jax.experimental import pallas as pl
from jax.experimental.pallas import tpu as pltpu
from jax.experimental.pallas import tpu_sc as plsc

_B = 16384
_RANK = 32
_SIGMA = 1.0
_LAMBD = 1.0 / 64.0

_NW = 32
_BPW = _B // _NW
_V0 = 1000000
_EDGE = 999936
_RNG = _EDGE // _NW
_PC = 512
_NCH = 63
_SCAP = 64
_P2 = 1009

_MANT = 0x007FFFFF
_ONEBITS = 0x3F800000
_SQRT2 = 1.41421356237
_LN2 = 0.6931471805599453

_LOGP = (7.0376836292e-2, -1.1514610310e-1, 1.1676998740e-1,
         -1.2420140846e-1, 1.4249322787e-1, -1.6668057665e-1,
         2.0000714765e-1, -2.4999993993e-1, 3.3333331174e-1)

_GDN = lax.GatherDimensionNumbers(
    offset_dims=(), collapsed_slice_dims=(0,), start_index_map=(0,))


def _permute(v, idx):
    return lax.gather(v, idx[:, None], dimension_numbers=_GDN,
                      slice_sizes=(1,),
                      mode=lax.GatherScatterMode.PROMISE_IN_BOUNDS)


def _hsum_all(v, lane):
    for k in (8, 4, 2, 1):
        v = v + _permute(v, lane ^ k)
    return v


def _full_log(x):
    iv = lax.bitcast_convert_type(x, jnp.int32)
    e = (iv >> 23) - 127
    m = lax.bitcast_convert_type((iv & _MANT) | _ONEBITS, jnp.float32)
    big = m > _SQRT2
    m = jnp.where(big, m * 0.5, m)
    e = jnp.where(big, e + 1, e)
    t = m - 1.0
    p = jnp.full((16,), _LOGP[0], jnp.float32)
    for c in _LOGP[1:]:
        p = p * t + c
    lnm = t + t * t * (t * p - 0.5)
    return lnm, e


def _sc_extract0(m0h, c0h, i0h, g0h,
                 idxb, lw, bm0a, bc0a, bm0b, bc0b, bm0c, bc0c,
                 stag, sidv, sem, semb, semc, ssem):
    nc = 2
    wid = lax.axis_index("s") * nc + lax.axis_index("c")
    base = wid * _RNG
    astart = (base // 128) * 128
    lane = lax.iota(jnp.int32, 16)

    def bin_piece(p, off):
        pltpu.sync_copy(i0h.at[pl.ds(p * 2048, 2048)], idxb)

        def bin_vreg(j, off):
            v = idxb[pl.ds(j * 16, 16)]
            col = v - base
            mask = (col >= 0) & (col < _RNG)
            ids = p * 2048 + j * 16 + lane
            pk = (col << 14) | ids
            plsc.store_compressed(lw.at[pl.ds(off, 16)], pk, mask=mask)
            return off + plsc.all_reduce_population_count(mask)[0]

        return lax.fori_loop(0, 128, bin_vreg, off)

    nw = lax.fori_loop(0, 8, bin_piece, jnp.int32(0))
    nv = (nw + 15) >> 4

    def fire(c, bm, bc, sem):
        cs = jnp.minimum(astart + c * _PC, _EDGE - _PC)
        pltpu.async_copy(m0h.at[:, pl.ds(cs, _PC)], bm, sem)
        pltpu.async_copy(c0h.at[:, pl.ds(cs, _PC)], bc, sem)

    def drain(bm, bc, sem):
        pltpu.make_async_copy(m0h.at[:, pl.ds(0, _PC)], bm, sem).wait()
        pltpu.make_async_copy(c0h.at[:, pl.ds(0, _PC)], bc, sem).wait()

    def flush(soff):
        def padv(k, _):
            sl = pl.ds(k * 16, 16)
            cur = sidv[sl]
            sidv[sl] = jnp.where(k * 16 + lane >= soff, _B, cur)
            return 0

        lax.fori_loop(0, _SCAP // 16, padv, 0)
        pltpu.async_copy(stag, g0h.at[sidv], ssem)
        pltpu.make_async_copy(stag, g0h.at[sidv], ssem).wait()

    def process_chunk(c, bm, bc, soff):
        cs = jnp.minimum(astart + c * _PC, _EDGE - _PC)

        def scan_vreg(j, soff):
            v = lw[pl.ds(j * 16, 16)]
            valid = (j * 16 + lane) < nw
            col = (v >> 14) + base - cs
            mask = valid & (col >= 0) & (col < _PC)
            cnt = plsc.all_reduce_population_count(mask)[0]
            mi = mask.astype(jnp.int32)

            def lane_work(soff):
                for i in range(16):
                    soff = _lane_extract(soff, v[i], col[i], mi[i],
                                         bm, bc, stag, sidv, lane, flush)
                return soff

            return lax.cond(cnt > 0, lane_work, lambda s: s, soff)

        return lax.fori_loop(0, nv, scan_vreg, soff)

    def _lane_extract(soff, pk, cl, hit, bm, bc, stag, sidv, lane, flush):
        def go(soff):
            eid = pk & 16383
            cc = jnp.full((16,), 1, jnp.int32) * cl
            m0a = plsc.load_gather(bm, [lane, cc])
            m0b = plsc.load_gather(bm, [lane + 16, cc])
            c0a = plsc.load_gather(bc, [lane, cc])
            c0b = plsc.load_gather(bc, [lane + 16, cc])
            stag[soff, pl.ds(0, 16)] = m0a
            stag[soff, pl.ds(16, 16)] = m0b
            stag[soff, pl.ds(32, 16)] = c0a
            stag[soff, pl.ds(48, 16)] = c0b
            sl = pl.ds((soff >> 4) * 16, 16)
            cur = sidv[sl]
            sidv[sl] = jnp.where(lane == (soff & 15), eid, cur)
            soff = soff + 1

            @pl.when(soff == _SCAP)
            def _():
                flush(jnp.int32(_SCAP))

            return jnp.where(soff == _SCAP, 0, soff)

        return lax.cond(hit == 1, go, lambda s: s, soff)

    rings = ((bm0a, bc0a, sem), (bm0b, bc0b, semb), (bm0c, bc0c, semc))
    fire(0, *rings[0])
    fire(1, *rings[1])

    def super_body(gi, soff):
        for b in (0, 1, 2):
            c = gi * 3 + b
            nxt = rings[(b + 2) % 3]

            @pl.when(c + 2 < _NCH)
            def _():
                fire(c + 2, *nxt)

            drain(*rings[b])
            soff = process_chunk(c, rings[b][0], rings[b][1], soff)
        return soff

    soff = lax.fori_loop(0, _NCH // 3, super_body, jnp.int32(0))

    @pl.when(soff > 0)
    def _():
        flush(soff)


def _sc_main(g0h, m1h, c1h, m2h, c2h, m0eh, c0eh,
             valsh, i0h, i1h, i2h, out_h,
             i0v, i1v, s1v, i2v, valsv,
             m2s, c2s, m0es, c0es,
             bg0a, bm1a, bc1a, bg0b, bm1b, bc1b,
             partv, sem0, sem1):
    nc = 2
    wid = lax.axis_index("s") * nc + lax.axis_index("c")
    base = wid * _BPW

    pltpu.sync_copy(i0h.at[pl.ds(base, _BPW)], i0v)
    pltpu.sync_copy(i1h.at[pl.ds(base, _BPW)], i1v)
    pltpu.sync_copy(i2h.at[pl.ds(base, _BPW)], i2v)
    pltpu.sync_copy(valsh.at[pl.ds(base, _BPW)], valsv)
    pltpu.sync_copy(m2h, m2s)
    pltpu.sync_copy(c2h, c2s)
    pltpu.sync_copy(m0eh, m0es)
    pltpu.sync_copy(c0eh, c0es)

    def shift_body(j, _):
        sl = pl.ds(j * 16, 16)
        s1v[sl] = i1v[sl] >> 2
        return 0

    lax.fori_loop(0, _BPW // 16, shift_body, 0)

    def fire(c, bg, bm, bc, sem):
        sl = pl.ds(c * 64, 64)
        pltpu.async_copy(g0h.at[pl.ds(base + c * 64, 64)], bg, sem)
        pltpu.async_copy(m1h.at[s1v.at[sl]], bm, sem)
        pltpu.async_copy(c1h.at[s1v.at[sl]], bc, sem)

    def drain(bg, bm, bc, sem):
        pltpu.make_async_copy(g0h.at[pl.ds(0, 64)], bg, sem).wait()
        pltpu.make_async_copy(m1h.at[s1v.at[pl.ds(0, 64)]], bm, sem).wait()
        pltpu.make_async_copy(c1h.at[s1v.at[pl.ds(0, 64)]], bc, sem).wait()

    lane = lax.iota(jnp.int32, 16)
    laneb = lane + 16
    zeros = jnp.zeros((16,), jnp.float32)
    ones = jnp.full((16,), 1.0, jnp.float32)

    def compute_chunk(c, bufs, accs):
        bg, bm1, bc1 = bufs

        def block_body(blk, carry):
            a_s1, a_m2, a_v, a_ln, a_e = carry
            gb = c * 64 + blk * 16
            i0b = i0v[pl.ds(gb, 16)]
            i1b = i1v[pl.ds(gb, 16)]
            i2b = i2v[pl.ds(gb, 16)]
            wv1 = (i1b & 3) * 32
            wv2 = (i2b & 3) * 32
            j2b = i2b >> 2
            predv = zeros
            pacc = ones
            for i in range(16):
                r = blk * 16 + i
                i0e = i0b[i]
                edge = (i0e >= _EDGE).astype(jnp.int32)
                em = (jnp.full((16,), 1, jnp.int32) * edge) == 1
                ec = jnp.full((16,), 1, jnp.int32) * jnp.clip(
                    i0e - _EDGE, 0, 63)
                w1 = wv1[i]
                w2 = wv2[i]
                j2 = j2b[i]
                m0a = jnp.where(em, plsc.load_gather(m0es, [lane, ec]),
                                bg[r, pl.ds(0, 16)])
                m0b = jnp.where(em, plsc.load_gather(m0es, [laneb, ec]),
                                bg[r, pl.ds(16, 16)])
                c0a = jnp.where(em, plsc.load_gather(c0es, [lane, ec]),
                                bg[r, pl.ds(32, 16)])
                c0b = jnp.where(em, plsc.load_gather(c0es, [laneb, ec]),
                                bg[r, pl.ds(48, 16)])
                m1a = bm1[r, pl.ds(w1, 16)]
                m1b = bm1[r, pl.ds(w1 + 16, 16)]
                c1a = bc1[r, pl.ds(w1, 16)]
                c1b = bc1[r, pl.ds(w1 + 16, 16)]
                m2a = m2s[j2, pl.ds(w2, 16)]
                m2b = m2s[j2, pl.ds(w2 + 16, 16)]
                c2a = c2s[j2, pl.ds(w2, 16)]
                c2b = c2s[j2, pl.ds(w2 + 16, 16)]
                ts = m0a * m1a * m2a + m0b * m1b * m2b
                predv = jnp.where(lane == i, _hsum_all(ts, lane), predv)
                a_m2 = (a_m2 + m0a * m0a + m0b * m0b + m1a * m1a
                        + m1b * m1b + m2a * m2a + m2b * m2b)
                for l in (c0a, c0b, c1a, c1b, c2a, c2b):
                    var = l * l
                    a_v = a_v + var
                    iv = lax.bitcast_convert_type(var, jnp.int32)
                    a_e = a_e + ((iv >> 23) - 127)
                    pacc = pacc * lax.bitcast_convert_type(
                        (iv & _MANT) | _ONEBITS, jnp.float32)
            q = valsv[pl.ds(gb, 16)]
            d = q - predv
            a_s1 = a_s1 + d * d
            lnm, e2 = _full_log(pacc)
            return (a_s1, a_m2, a_v, a_ln + lnm, a_e + e2)

        return lax.fori_loop(0, 4, block_body, accs)

    bufsa = (bg0a, bm1a, bc1a)
    bufsb = (bg0b, bm1b, bc1b)
    accs = (zeros, zeros, zeros, zeros, jnp.zeros((16,), jnp.int32))
    fire(0, *bufsa, sem0)

    def super_body(gi, accs):
        for b in (0, 1):
            c = gi * 2 + b
            bufs, sem = (bufsa, sem0) if b == 0 else (bufsb, sem1)
            nbufs, nsem = (bufsb, sem1) if b == 0 else (bufsa, sem0)

            @pl.when(c + 1 < _BPW // 64)
            def _():
                fire(c + 1, *nbufs, nsem)

            drain(*bufs, sem)
            accs = compute_chunk(c, bufs, accs)
        return accs

    a_s1, a_m2, a_v, a_ln, a_e = lax.fori_loop(0, _BPW // 128, super_body,
                                               accs)

    partv[pl.ds(0, 16)] = a_s1
    partv[pl.ds(16, 16)] = a_m2
    partv[pl.ds(32, 16)] = a_v
    partv[pl.ds(48, 16)] = a_ln
    partv[pl.ds(64, 16)] = a_e.astype(jnp.float32)
    pltpu.sync_copy(partv, out_h.at[wid])


def _combine_body(parts_ref, o_ref):
    import math
    const = (0.5 * _B * math.log(2.0 * math.pi * _SIGMA ** 2)
             - 0.5 * _LAMBD * 3.0 * _B * _RANK)
    p = parts_ref[...]
    total = (0.5 / (_SIGMA ** 2) * jnp.sum(p[:, 0:16])
             + 0.5 * _LAMBD * jnp.sum(p[:, 16:32])
             + 0.5 * _LAMBD * jnp.sum(p[:, 32:48])
             - 0.5 * _LAMBD * jnp.sum(p[:, 48:64])
             - 0.5 * _LAMBD * _LN2 * jnp.sum(p[:, 64:80])
             + const)
    o_ref[...] = jnp.reshape(total, (1, 1))


def kernel(mean0, mean1, mean2, chol0, chol1, chol2, vals, idx0, idx1, idx2):
    mesh = plsc.VectorSubcoreMesh(core_axis_name="c", subcore_axis_name="s")
    cp = pltpu.CompilerParams(needs_layout_passes=False)

    extract = functools.partial(
        pl.kernel, mesh=mesh, compiler_params=cp,
        out_type=jax.ShapeDtypeStruct((_B + 1, 128), jnp.float32),
        scratch_types=[
            pltpu.VMEM((2048,), jnp.int32),
            pltpu.VMEM((_B,), jnp.int32),
            pltpu.VMEM((_RANK, _PC), jnp.float32),
            pltpu.VMEM((_RANK, _PC), jnp.float32),
            pltpu.VMEM((_RANK, _PC), jnp.float32),
            pltpu.VMEM((_RANK, _PC), jnp.float32),
            pltpu.VMEM((_RANK, _PC), jnp.float32),
            pltpu.VMEM((_RANK, _PC), jnp.float32),
            pltpu.VMEM((_SCAP, 128), jnp.float32),
            pltpu.VMEM((_SCAP,), jnp.int32),
            pltpu.SemaphoreType.DMA,
            pltpu.SemaphoreType.DMA,
            pltpu.SemaphoreType.DMA,
            pltpu.SemaphoreType.DMA,
        ],
    )(_sc_extract0)
    g0 = extract(mean0.T, chol0.T, idx0)

    main = functools.partial(
        pl.kernel, mesh=mesh, compiler_params=cp,
        out_type=jax.ShapeDtypeStruct((_NW, 80), jnp.float32),
        scratch_types=[
            pltpu.VMEM((_BPW,), jnp.int32),
            pltpu.VMEM((_BPW,), jnp.int32),
            pltpu.VMEM((_BPW,), jnp.int32),
            pltpu.VMEM((_BPW,), jnp.int32),
            pltpu.VMEM((_BPW,), jnp.float32),
            pltpu.VMEM((250, 128), jnp.float32),
            pltpu.VMEM((250, 128), jnp.float32),
            pltpu.VMEM((_RANK, 64), jnp.float32),
            pltpu.VMEM((_RANK, 64), jnp.float32),
            pltpu.VMEM((64, 128), jnp.float32),
            pltpu.VMEM((64, 128), jnp.float32),
            pltpu.VMEM((64, 128), jnp.float32),
            pltpu.VMEM((64, 128), jnp.float32),
            pltpu.VMEM((64, 128), jnp.float32),
            pltpu.VMEM((64, 128), jnp.float32),
            pltpu.VMEM((80,), jnp.float32),
            pltpu.SemaphoreType.DMA,
            pltpu.SemaphoreType.DMA,
        ],
    )(_sc_main)
    parts = main(g0, mean1.reshape(-1, 128), chol1.reshape(-1, 128),
                 mean2.reshape(-1, 128), chol2.reshape(-1, 128),
                 mean0.T[:, _EDGE:], chol0.T[:, _EDGE:],
                 vals, idx0, idx1, idx2)
    out = pl.pallas_call(
        _combine_body,
        out_shape=jax.ShapeDtypeStruct((1, 1), jnp.float32),
    )(parts)
    return out[0, 0]


# speedup vs baseline: 1.0859x; 1.0859x over previous
import functools

import jax
import jax.numpy as jnp
from jax import lax
from jax.experimental import pallas as pl
from jax.experimental.pallas import tpu as pltpu
from jax.experimental.pallas import tpu_sc as plsc

_B = 16384
_RANK = 32
_SIGMA = 1.0
_LAMBD = 1.0 / 64.0

_NW = 32
_BPW = _B // _NW
_V0 = 1000000
_EDGE = 999936
_RNG = _EDGE // _NW
_PC = 768
_NCH = 42
_SCAP = 64
_P2 = 1009

_MANT = 0x007FFFFF
_ONEBITS = 0x3F800000
_SQRT2 = 1.41421356237
_LN2 = 0.6931471805599453

_LOGP = (7.0376836292e-2, -1.1514610310e-1, 1.1676998740e-1,
         -1.2420140846e-1, 1.4249322787e-1, -1.6668057665e-1,
         2.0000714765e-1, -2.4999993993e-1, 3.3333331174e-1)

_GDN = lax.GatherDimensionNumbers(
    offset_dims=(), collapsed_slice_dims=(0,), start_index_map=(0,))


def _permute(v, idx):
    return lax.gather(v, idx[:, None], dimension_numbers=_GDN,
                      slice_sizes=(1,),
                      mode=lax.GatherScatterMode.PROMISE_IN_BOUNDS)


def _hsum_all(v, lane):
    for k in (8, 4, 2, 1):
        v = v + _permute(v, lane ^ k)
    return v


def _full_log(x):
    iv = lax.bitcast_convert_type(x, jnp.int32)
    e = (iv >> 23) - 127
    m = lax.bitcast_convert_type((iv & _MANT) | _ONEBITS, jnp.float32)
    big = m > _SQRT2
    m = jnp.where(big, m * 0.5, m)
    e = jnp.where(big, e + 1, e)
    t = m - 1.0
    p = jnp.full((16,), _LOGP[0], jnp.float32)
    for c in _LOGP[1:]:
        p = p * t + c
    lnm = t + t * t * (t * p - 0.5)
    return lnm, e


def _sc_extract0(m0h, c0h, i0h, g0h,
                 idxb, lw, bm0a, bc0a, bm0b, bc0b,
                 stag, sidv, sem, semb, ssem):
    nc = 2
    wid = lax.axis_index("s") * nc + lax.axis_index("c")
    base = wid * _RNG
    astart = (base // 128) * 128
    lane = lax.iota(jnp.int32, 16)

    def bin_piece(p, off):
        pltpu.sync_copy(i0h.at[pl.ds(p * 2048, 2048)], idxb)

        def bin_vreg(j, off):
            v = idxb[pl.ds(j * 16, 16)]
            col = v - base
            mask = (col >= 0) & (col < _RNG)
            ids = p * 2048 + j * 16 + lane
            pk = (col << 14) | ids
            plsc.store_compressed(lw.at[pl.ds(off, 16)], pk, mask=mask)
            return off + plsc.all_reduce_population_count(mask)[0]

        return lax.fori_loop(0, 128, bin_vreg, off)

    nw = lax.fori_loop(0, 8, bin_piece, jnp.int32(0))
    nv = (nw + 15) >> 4

    def fire(c, bm, bc, sem):
        cs = jnp.minimum(astart + c * _PC, _EDGE - _PC)
        pltpu.async_copy(m0h.at[:, pl.ds(cs, _PC)], bm, sem)
        pltpu.async_copy(c0h.at[:, pl.ds(cs, _PC)], bc, sem)

    def drain(bm, bc, sem):
        pltpu.make_async_copy(m0h.at[:, pl.ds(0, _PC)], bm, sem).wait()
        pltpu.make_async_copy(c0h.at[:, pl.ds(0, _PC)], bc, sem).wait()

    def flush(soff):
        def padv(k, _):
            sl = pl.ds(k * 16, 16)
            cur = sidv[sl]
            sidv[sl] = jnp.where(k * 16 + lane >= soff, _B, cur)
            return 0

        lax.fori_loop(0, _SCAP // 16, padv, 0)
        pltpu.async_copy(stag, g0h.at[sidv], ssem)
        pltpu.make_async_copy(stag, g0h.at[sidv], ssem).wait()

    def process_chunk(c, bm, bc, soff):
        cs = jnp.minimum(astart + c * _PC, _EDGE - _PC)

        def scan_vreg(j, soff):
            v = lw[pl.ds(j * 16, 16)]
            valid = (j * 16 + lane) < nw
            col = (v >> 14) + base - cs
            mask = valid & (col >= 0) & (col < _PC)
            cnt = plsc.all_reduce_population_count(mask)[0]
            mi = mask.astype(jnp.int32)

            def lane_work(soff):
                for i in range(16):
                    soff = _lane_extract(soff, v[i], col[i], mi[i],
                                         bm, bc, stag, sidv, lane, flush)
                return soff

            return lax.cond(cnt > 0, lane_work, lambda s: s, soff)

        return lax.fori_loop(0, nv, scan_vreg, soff)

    def _lane_extract(soff, pk, cl, hit, bm, bc, stag, sidv, lane, flush):
        def go(soff):
            eid = pk & 16383
            cc = jnp.full((16,), 1, jnp.int32) * cl
            m0a = plsc.load_gather(bm, [lane, cc])
            m0b = plsc.load_gather(bm, [lane + 16, cc])
            c0a = plsc.load_gather(bc, [lane, cc])
            c0b = plsc.load_gather(bc, [lane + 16, cc])
            stag[soff, pl.ds(0, 16)] = m0a
            stag[soff, pl.ds(16, 16)] = m0b
            stag[soff, pl.ds(32, 16)] = c0a
            stag[soff, pl.ds(48, 16)] = c0b
            sl = pl.ds((soff >> 4) * 16, 16)
            cur = sidv[sl]
            sidv[sl] = jnp.where(lane == (soff & 15), eid, cur)
            soff = soff + 1

            @pl.when(soff == _SCAP)
            def _():
                flush(jnp.int32(_SCAP))

            return jnp.where(soff == _SCAP, 0, soff)

        return lax.cond(hit == 1, go, lambda s: s, soff)

    rings = ((bm0a, bc0a, sem), (bm0b, bc0b, semb))
    fire(0, *rings[0])

    def super_body(gi, soff):
        for b in (0, 1):
            c = gi * 2 + b
            nxt = rings[(b + 1) % 2]

            @pl.when(c + 1 < _NCH)
            def _():
                fire(c + 1, *nxt)

            drain(*rings[b])
            soff = process_chunk(c, rings[b][0], rings[b][1], soff)
        return soff

    soff = lax.fori_loop(0, _NCH // 2, super_body, jnp.int32(0))

    @pl.when(soff > 0)
    def _():
        flush(soff)


def _sc_main(g0h, m1h, c1h, m2h, c2h, m0eh, c0eh,
             valsh, i0h, i1h, i2h, out_h,
             i0v, i1v, s1v, i2v, valsv,
             m2s, c2s, m0es, c0es,
             bg0a, bm1a, bc1a, bg0b, bm1b, bc1b,
             partv, sem0, sem1):
    nc = 2
    wid = lax.axis_index("s") * nc + lax.axis_index("c")
    base = wid * _BPW

    pltpu.sync_copy(i0h.at[pl.ds(base, _BPW)], i0v)
    pltpu.sync_copy(i1h.at[pl.ds(base, _BPW)], i1v)
    pltpu.sync_copy(i2h.at[pl.ds(base, _BPW)], i2v)
    pltpu.sync_copy(valsh.at[pl.ds(base, _BPW)], valsv)
    pltpu.sync_copy(m2h, m2s)
    pltpu.sync_copy(c2h, c2s)
    pltpu.sync_copy(m0eh, m0es)
    pltpu.sync_copy(c0eh, c0es)

    def shift_body(j, _):
        sl = pl.ds(j * 16, 16)
        s1v[sl] = i1v[sl] >> 2
        return 0

    lax.fori_loop(0, _BPW // 16, shift_body, 0)

    def fire(c, bg, bm, bc, sem):
        sl = pl.ds(c * 64, 64)
        pltpu.async_copy(g0h.at[pl.ds(base + c * 64, 64)], bg, sem)
        pltpu.async_copy(m1h.at[s1v.at[sl]], bm, sem)
        pltpu.async_copy(c1h.at[s1v.at[sl]], bc, sem)

    def drain(bg, bm, bc, sem):
        pltpu.make_async_copy(g0h.at[pl.ds(0, 64)], bg, sem).wait()
        pltpu.make_async_copy(m1h.at[s1v.at[pl.ds(0, 64)]], bm, sem).wait()
        pltpu.make_async_copy(c1h.at[s1v.at[pl.ds(0, 64)]], bc, sem).wait()

    lane = lax.iota(jnp.int32, 16)
    laneb = lane + 16
    zeros = jnp.zeros((16,), jnp.float32)
    ones = jnp.full((16,), 1.0, jnp.float32)

    def compute_chunk(c, bufs, accs):
        bg, bm1, bc1 = bufs

        def block_body(blk, carry):
            a_s1, a_m2, a_v, a_ln, a_e = carry
            gb = c * 64 + blk * 16
            i0b = i0v[pl.ds(gb, 16)]
            i1b = i1v[pl.ds(gb, 16)]
            i2b = i2v[pl.ds(gb, 16)]
            wv1 = (i1b & 3) * 32
            wv2 = (i2b & 3) * 32
            j2b = i2b >> 2
            predv = zeros
            pacc = ones
            for i in range(16):
                r = blk * 16 + i
                i0e = i0b[i]
                edge = (i0e >= _EDGE).astype(jnp.int32)
                em = (jnp.full((16,), 1, jnp.int32) * edge) == 1
                ec = jnp.full((16,), 1, jnp.int32) * jnp.clip(
                    i0e - _EDGE, 0, 63)
                w1 = wv1[i]
                w2 = wv2[i]
                j2 = j2b[i]
                m0a = jnp.where(em, plsc.load_gather(m0es, [lane, ec]),
                                bg[r, pl.ds(0, 16)])
                m0b = jnp.where(em, plsc.load_gather(m0es, [laneb, ec]),
                                bg[r, pl.ds(16, 16)])
                c0a = jnp.where(em, plsc.load_gather(c0es, [lane, ec]),
                                bg[r, pl.ds(32, 16)])
                c0b = jnp.where(em, plsc.load_gather(c0es, [laneb, ec]),
                                bg[r, pl.ds(48, 16)])
                m1a = bm1[r, pl.ds(w1, 16)]
                m1b = bm1[r, pl.ds(w1 + 16, 16)]
                c1a = bc1[r, pl.ds(w1, 16)]
                c1b = bc1[r, pl.ds(w1 + 16, 16)]
                m2a = m2s[j2, pl.ds(w2, 16)]
                m2b = m2s[j2, pl.ds(w2 + 16, 16)]
                c2a = c2s[j2, pl.ds(w2, 16)]
                c2b = c2s[j2, pl.ds(w2 + 16, 16)]
                ts = m0a * m1a * m2a + m0b * m1b * m2b
                predv = jnp.where(lane == i, _hsum_all(ts, lane), predv)
                a_m2 = (a_m2 + m0a * m0a + m0b * m0b + m1a * m1a
                        + m1b * m1b + m2a * m2a + m2b * m2b)
                for l in (c0a, c0b, c1a, c1b, c2a, c2b):
                    var = l * l
                    a_v = a_v + var
                    iv = lax.bitcast_convert_type(var, jnp.int32)
                    a_e = a_e + ((iv >> 23) - 127)
                    pacc = pacc * lax.bitcast_convert_type(
                        (iv & _MANT) | _ONEBITS, jnp.float32)
            q = valsv[pl.ds(gb, 16)]
            d = q - predv
            a_s1 = a_s1 + d * d
            lnm, e2 = _full_log(pacc)
            return (a_s1, a_m2, a_v, a_ln + lnm, a_e + e2)

        return lax.fori_loop(0, 4, block_body, accs)

    bufsa = (bg0a, bm1a, bc1a)
    bufsb = (bg0b, bm1b, bc1b)
    accs = (zeros, zeros, zeros, zeros, jnp.zeros((16,), jnp.int32))
    fire(0, *bufsa, sem0)

    def super_body(gi, accs):
        for b in (0, 1):
            c = gi * 2 + b
            bufs, sem = (bufsa, sem0) if b == 0 else (bufsb, sem1)
            nbufs, nsem = (bufsb, sem1) if b == 0 else (bufsa, sem0)

            @pl.when(c + 1 < _BPW // 64)
            def _():
                fire(c + 1, *nbufs, nsem)

            drain(*bufs, sem)
            accs = compute_chunk(c, bufs, accs)
        return accs

    a_s1, a_m2, a_v, a_ln, a_e = lax.fori_loop(0, _BPW // 128, super_body,
                                               accs)

    partv[pl.ds(0, 16)] = a_s1
    partv[pl.ds(16, 16)] = a_m2
    partv[pl.ds(32, 16)] = a_v
    partv[pl.ds(48, 16)] = a_ln
    partv[pl.ds(64, 16)] = a_e.astype(jnp.float32)
    pltpu.sync_copy(partv, out_h.at[wid])


def _combine_body(parts_ref, o_ref):
    import math
    const = (0.5 * _B * math.log(2.0 * math.pi * _SIGMA ** 2)
             - 0.5 * _LAMBD * 3.0 * _B * _RANK)
    p = parts_ref[...]
    total = (0.5 / (_SIGMA ** 2) * jnp.sum(p[:, 0:16])
             + 0.5 * _LAMBD * jnp.sum(p[:, 16:32])
             + 0.5 * _LAMBD * jnp.sum(p[:, 32:48])
             - 0.5 * _LAMBD * jnp.sum(p[:, 48:64])
             - 0.5 * _LAMBD * _LN2 * jnp.sum(p[:, 64:80])
             + const)
    o_ref[...] = jnp.reshape(total, (1, 1))


def kernel(mean0, mean1, mean2, chol0, chol1, chol2, vals, idx0, idx1, idx2):
    mesh = plsc.VectorSubcoreMesh(core_axis_name="c", subcore_axis_name="s")
    cp = pltpu.CompilerParams(needs_layout_passes=False)

    extract = functools.partial(
        pl.kernel, mesh=mesh, compiler_params=cp,
        out_type=jax.ShapeDtypeStruct((_B + 1, 128), jnp.float32),
        scratch_types=[
            pltpu.VMEM((2048,), jnp.int32),
            pltpu.VMEM((_B,), jnp.int32),
            pltpu.VMEM((_RANK, _PC), jnp.float32),
            pltpu.VMEM((_RANK, _PC), jnp.float32),
            pltpu.VMEM((_RANK, _PC), jnp.float32),
            pltpu.VMEM((_RANK, _PC), jnp.float32),
            pltpu.VMEM((_SCAP, 128), jnp.float32),
            pltpu.VMEM((_SCAP,), jnp.int32),
            pltpu.SemaphoreType.DMA,
            pltpu.SemaphoreType.DMA,
            pltpu.SemaphoreType.DMA,
        ],
    )(_sc_extract0)
    g0 = extract(mean0.T, chol0.T, idx0)

    main = functools.partial(
        pl.kernel, mesh=mesh, compiler_params=cp,
        out_type=jax.ShapeDtypeStruct((_NW, 80), jnp.float32),
        scratch_types=[
            pltpu.VMEM((_BPW,), jnp.int32),
            pltpu.VMEM((_BPW,), jnp.int32),
            pltpu.VMEM((_BPW,), jnp.int32),
            pltpu.VMEM((_BPW,), jnp.int32),
            pltpu.VMEM((_BPW,), jnp.float32),
            pltpu.VMEM((250, 128), jnp.float32),
            pltpu.VMEM((250, 128), jnp.float32),
            pltpu.VMEM((_RANK, 64), jnp.float32),
            pltpu.VMEM((_RANK, 64), jnp.float32),
            pltpu.VMEM((64, 128), jnp.float32),
            pltpu.VMEM((64, 128), jnp.float32),
            pltpu.VMEM((64, 128), jnp.float32),
            pltpu.VMEM((64, 128), jnp.float32),
            pltpu.VMEM((64, 128), jnp.float32),
            pltpu.VMEM((64, 128), jnp.float32),
            pltpu.VMEM((80,), jnp.float32),
            pltpu.SemaphoreType.DMA,
            pltpu.SemaphoreType.DMA,
        ],
    )(_sc_main)
    parts = main(g0, mean1.reshape(-1, 128), chol1.reshape(-1, 128),
                 mean2.reshape(-1, 128), chol2.reshape(-1, 128),
                 mean0.T[:, _EDGE:], chol0.T[:, _EDGE:],
                 vals, idx0, idx1, idx2)
    out = pl.pallas_call(
        _combine_body,
        out_shape=jax.ShapeDtypeStruct((1, 1), jnp.float32),
    )(parts)
    return out[0, 0]
